# baseline (speedup 1.0000x reference)
import jax
import jax.numpy as jnp
from jax.experimental import pallas as pl
from jax.experimental.pallas import tpu as pltpu

N = 10000
F = 128
C = 64
ROWS = 200
NB = N // ROWS


def _fused_kernel(adj_ref, x_ref, w_in_ref, b_in_ref, b0_ref, w1_ref,
                  b1_ref, w_out_ref, b_out_ref, o_ref, t0_ref, t1_ref):
    s = pl.program_id(0)

    @pl.when(s == 0)
    def _precompute_t0():
        t0_ref[...] = (
            jnp.dot(x_ref[...], w_in_ref[...],
                    preferred_element_type=jnp.float32)
            + b_in_ref[...]
        )

    @pl.when((s >= 1) & (s <= NB))
    def _pass1():
        i = s - 1
        acc = jnp.dot(adj_ref[...], t0_ref[...],
                      preferred_element_type=jnp.float32)
        h = jnp.maximum(acc + b0_ref[...], 0.0)
        t1_ref[pl.ds(i * ROWS, ROWS), :] = jnp.dot(
            h, w1_ref[...], preferred_element_type=jnp.float32
        )

    @pl.when(s > NB)
    def _pass2():
        acc = jnp.dot(adj_ref[...], t1_ref[...],
                      preferred_element_type=jnp.float32)
        h = jnp.maximum(acc + b1_ref[...], 0.0)
        o_ref[...] = (
            jnp.dot(h, w_out_ref[...], preferred_element_type=jnp.float32)
            + b_out_ref[...]
        )


def _adj_index(s):
    return (jnp.where(s == 0, 0, (s - 1) % NB), 0)


def _out_index(s):
    return (jnp.where(s <= NB, 0, s - (NB + 1)), 0)


def kernel(x, adj, bn_gamma, bn_beta, fc_in_w, fc_in_b,
           gc0_w, gc0_b, gc1_w, gc1_b, fc_out_w, fc_out_b):
    eps = 1e-5
    scale = bn_gamma / jnp.sqrt(1.0 + eps)
    w_in = (scale[:, None] * fc_in_w.T) @ gc0_w
    b_in = ((bn_beta @ fc_in_w.T + fc_in_b) @ gc0_w)[None, :]

    grid = (2 * NB + 1,)
    return pl.pallas_call(
        _fused_kernel,
        grid=grid,
        in_specs=[
            pl.BlockSpec((ROWS, N), _adj_index),
            pl.BlockSpec((N, F), lambda s: (0, 0)),
            pl.BlockSpec((F, F), lambda s: (0, 0)),
            pl.BlockSpec((1, F), lambda s: (0, 0)),
            pl.BlockSpec((1, F), lambda s: (0, 0)),
            pl.BlockSpec((F, F), lambda s: (0, 0)),
            pl.BlockSpec((1, F), lambda s: (0, 0)),
            pl.BlockSpec((F, C), lambda s: (0, 0)),
            pl.BlockSpec((1, C), lambda s: (0, 0)),
        ],
        out_specs=pl.BlockSpec((ROWS, C), _out_index),
        out_shape=jax.ShapeDtypeStruct((N, C), jnp.float32),
        scratch_shapes=[
            pltpu.VMEM((N, F), jnp.float32),
            pltpu.VMEM((N, F), jnp.float32),
        ],
        compiler_params=pltpu.CompilerParams(
            dimension_semantics=("arbitrary",)
        ),
    )(adj, x, w_in, b_in, gc0_b, gc1_w, gc1_b, fc_out_w.T, fc_out_b[None, :])


# speedup vs baseline: 1.0342x; 1.0342x over previous
import jax
import jax.numpy as jnp
from jax.experimental import pallas as pl
from jax.experimental.pallas import tpu as pltpu

N = 10000
F = 128
C = 64
ROWS = 400
NB = N // ROWS


def _fused_kernel(adj_ref, x_ref, w_in_ref, b_in_ref, b0_ref, w1_ref,
                  b1_ref, w_out_ref, b_out_ref, o_ref, t0_ref, t1_ref):
    s = pl.program_id(0)

    @pl.when(s == 0)
    def _precompute_t0():
        t0_ref[...] = (
            jnp.dot(x_ref[...], w_in_ref[...],
                    preferred_element_type=jnp.float32)
            + b_in_ref[...]
        )

    @pl.when((s >= 1) & (s <= NB))
    def _pass1():
        i = s - 1
        acc = jnp.dot(adj_ref[...], t0_ref[...],
                      preferred_element_type=jnp.float32)
        h = jnp.maximum(acc + b0_ref[...], 0.0)
        t1_ref[pl.ds(i * ROWS, ROWS), :] = jnp.dot(
            h, w1_ref[...], preferred_element_type=jnp.float32
        )

    @pl.when(s > NB)
    def _pass2():
        acc = jnp.dot(adj_ref[...], t1_ref[...],
                      preferred_element_type=jnp.float32)
        h = jnp.maximum(acc + b1_ref[...], 0.0)
        o_ref[...] = (
            jnp.dot(h, w_out_ref[...], preferred_element_type=jnp.float32)
            + b_out_ref[...]
        )


def _adj_index(s):
    return (jnp.where(s == 0, 0,
                      jnp.where(s <= NB, s - 1, 2 * NB - s)), 0)


def _out_index(s):
    return (jnp.where(s <= NB, NB - 1, 2 * NB - s), 0)


def kernel(x, adj, bn_gamma, bn_beta, fc_in_w, fc_in_b,
           gc0_w, gc0_b, gc1_w, gc1_b, fc_out_w, fc_out_b):
    eps = 1e-5
    scale = bn_gamma / jnp.sqrt(1.0 + eps)
    w_in = (scale[:, None] * fc_in_w.T) @ gc0_w
    b_in = ((bn_beta @ fc_in_w.T + fc_in_b) @ gc0_w)[None, :]

    grid = (2 * NB + 1,)
    return pl.pallas_call(
        _fused_kernel,
        grid=grid,
        in_specs=[
            pl.BlockSpec((ROWS, N), _adj_index),
            pl.BlockSpec((N, F), lambda s: (0, 0)),
            pl.BlockSpec((F, F), lambda s: (0, 0)),
            pl.BlockSpec((1, F), lambda s: (0, 0)),
            pl.BlockSpec((1, F), lambda s: (0, 0)),
            pl.BlockSpec((F, F), lambda s: (0, 0)),
            pl.BlockSpec((1, F), lambda s: (0, 0)),
            pl.BlockSpec((F, C), lambda s: (0, 0)),
            pl.BlockSpec((1, C), lambda s: (0, 0)),
        ],
        out_specs=pl.BlockSpec((ROWS, C), _out_index),
        out_shape=jax.ShapeDtypeStruct((N, C), jnp.float32),
        scratch_shapes=[
            pltpu.VMEM((N, F), jnp.float32),
            pltpu.VMEM((N, F), jnp.float32),
        ],
        compiler_params=pltpu.CompilerParams(
            dimension_semantics=("arbitrary",)
        ),
    )(adj, x, w_in, b_in, gc0_b, gc1_w, gc1_b, fc_out_w.T, fc_out_b[None, :])


# speedup vs baseline: 1.0561x; 1.0212x over previous
import jax
import jax.numpy as jnp
from jax.experimental import pallas as pl
from jax.experimental.pallas import tpu as pltpu

N = 10000
F = 128
C = 64
ROWS = 400
NB = N // ROWS


def _fused_kernel(adj_ref, x_ref, bn_gamma_ref, bn_beta_ref, fc_in_w_ref,
                  fc_in_b_ref, gc0_w_ref, b0_ref, w1_ref, b1_ref,
                  fc_out_w_ref, b_out_ref, o_ref, t0_ref, t1_ref, w_out_ref):
    s = pl.program_id(0)

    @pl.when(s == 0)
    def _fold_weights_and_t0():
        eps = 1e-5
        scale = bn_gamma_ref[...] / jnp.sqrt(1.0 + eps)
        f_t = fc_in_w_ref[...].T
        w_in = jnp.dot(scale.T * f_t, gc0_w_ref[...],
                       preferred_element_type=jnp.float32)
        b_in = jnp.dot(
            jnp.dot(bn_beta_ref[...], f_t,
                    preferred_element_type=jnp.float32) + fc_in_b_ref[...],
            gc0_w_ref[...], preferred_element_type=jnp.float32)
        t0_ref[...] = (
            jnp.dot(x_ref[...], w_in, preferred_element_type=jnp.float32)
            + b_in
        )
        w_out_ref[...] = fc_out_w_ref[...].T

    @pl.when((s >= 1) & (s <= NB))
    def _pass1():
        i = s - 1
        acc = jnp.dot(adj_ref[...], t0_ref[...],
                      preferred_element_type=jnp.float32)
        h = jnp.maximum(acc + b0_ref[...], 0.0)
        t1_ref[pl.ds(i * ROWS, ROWS), :] = jnp.dot(
            h, w1_ref[...], preferred_element_type=jnp.float32
        )

    @pl.when(s > NB)
    def _pass2():
        acc = jnp.dot(adj_ref[...], t1_ref[...],
                      preferred_element_type=jnp.float32)
        h = jnp.maximum(acc + b1_ref[...], 0.0)
        o_ref[...] = (
            jnp.dot(h, w_out_ref[...], preferred_element_type=jnp.float32)
            + b_out_ref[...]
        )


def _adj_index(s):
    return (jnp.where(s == 0, 0,
                      jnp.where(s <= NB, s - 1, 2 * NB - s)), 0)


def _out_index(s):
    return (jnp.where(s <= NB, NB - 1, 2 * NB - s), 0)


def kernel(x, adj, bn_gamma, bn_beta, fc_in_w, fc_in_b,
           gc0_w, gc0_b, gc1_w, gc1_b, fc_out_w, fc_out_b):
    grid = (2 * NB + 1,)
    _res = lambda bs: pl.BlockSpec(bs, lambda s: (0, 0))
    return pl.pallas_call(
        _fused_kernel,
        grid=grid,
        in_specs=[
            pl.BlockSpec((ROWS, N), _adj_index),
            _res((N, F)),
            _res((1, F)),
            _res((1, F)),
            _res((F, F)),
            _res((1, F)),
            _res((F, F)),
            _res((1, F)),
            _res((F, F)),
            _res((1, F)),
            _res((C, F)),
            _res((1, C)),
        ],
        out_specs=pl.BlockSpec((ROWS, C), _out_index),
        out_shape=jax.ShapeDtypeStruct((N, C), jnp.float32),
        scratch_shapes=[
            pltpu.VMEM((N, F), jnp.float32),
            pltpu.VMEM((N, F), jnp.float32),
            pltpu.VMEM((F, C), jnp.float32),
        ],
        compiler_params=pltpu.CompilerParams(
            dimension_semantics=("arbitrary",)
        ),
    )(adj, x, bn_gamma[None, :], bn_beta[None, :], fc_in_w,
      fc_in_b[None, :], gc0_w, gc0_b, gc1_w, gc1_b, fc_out_w,
      fc_out_b[None, :])
